# Initial kernel scaffold; baseline (speedup 1.0000x reference)
import jax
import jax.numpy as jnp
from jax.experimental import pallas as pl


def kernel(x, params):
    raise NotImplementedError("write your pallas kernel here")


# speedup vs baseline: 1.6542x; 1.6542x over previous
import functools

import jax
import jax.numpy as jnp
from jax.experimental import pallas as pl
from jax.experimental.pallas import tpu as pltpu

NUM_CLASSES = 10
NUM_BLOCKS = 6
DIM = 512
HID = 2048
E = 8
TOPK = 2
BATCH = 512

INTERPRET = False


def _gelu(x):
    return 0.5 * x * (1.0 + jax.lax.erf(x * 0.7071067811865476))


def _ln(x, g, b, eps=1e-5):
    m = jnp.mean(x, axis=-1, keepdims=True)
    v = jnp.mean((x - m) ** 2, axis=-1, keepdims=True)
    return (x - m) * jax.lax.rsqrt(v + eps) * g + b


def _row(ref, idx):
    return ref[pl.ds(idx, 1)][0]


def _moe_kernel(h0, lng, lnb, rw, rb, b1r, b2r, hlng, hlnb, hw, hb,
                w1, w2, out, aux,
                h_scr, hn_scr, wts_scr, acc_scr, aux_scr):
    i = pl.program_id(0)
    e = pl.program_id(1)

    @pl.when((i == 0) & (e == 0))
    def _init():
        h_scr[...] = h0[...]
        aux_scr[...] = jnp.zeros((1, 1), jnp.float32)

    @pl.when(e == 0)
    def _router():
        h = h_scr[...]
        g = _row(lng, i)
        bb = _row(lnb, i)
        hn = _ln(h, g, bb)
        hn_scr[...] = hn
        rwi = _row(rw, i)
        rbi = _row(rb, i)
        logits = jax.lax.dot_general(
            hn, rwi, (((1,), (1,)), ((), ())),
            preferred_element_type=jnp.float32) + rbi
        mx = jnp.max(logits, axis=1, keepdims=True)
        ex = jnp.exp(logits - mx)
        probs = ex / jnp.sum(ex, axis=1, keepdims=True)
        eidx = jax.lax.broadcasted_iota(jnp.int32, (BATCH, E), 1)
        v1 = jnp.max(probs, axis=1, keepdims=True)
        i1 = jnp.min(jnp.where(probs == v1, eidx, E), axis=1, keepdims=True)
        m1 = eidx == i1
        p2 = jnp.where(m1, -1.0, probs)
        v2 = jnp.max(p2, axis=1, keepdims=True)
        i2 = jnp.min(jnp.where(p2 == v2, eidx, E), axis=1, keepdims=True)
        m2 = eidx == i2
        s = v1 + v2 + 1e-9
        wts = (v1 / s) * m1.astype(jnp.float32) + (v2 / s) * m2.astype(jnp.float32)
        wts_scr[...] = wts
        onehot = m1.astype(jnp.float32) + m2.astype(jnp.float32)
        f = jnp.mean(onehot, axis=0, keepdims=True)
        imp = jnp.mean(probs, axis=0, keepdims=True)
        aux_scr[...] += jnp.reshape((E / TOPK) * jnp.sum(f * imp), (1, 1))
        acc_scr[...] = jnp.zeros_like(acc_scr)

    hn = hn_scr[...]
    w1b = w1[0, 0]
    h1 = jax.lax.dot_general(hn, w1b, (((1,), (1,)), ((), ())),
                             preferred_element_type=jnp.float32)
    h1 = _gelu(h1 + _row(b1r, i * E + e))
    w2b = w2[0, 0]
    h2 = jax.lax.dot_general(h1, w2b, (((1,), (1,)), ((), ())),
                             preferred_element_type=jnp.float32)
    h2 = h2 + _row(b2r, i * E + e)
    eidx = jax.lax.broadcasted_iota(jnp.int32, (BATCH, E), 1)
    wcol = jnp.sum(jnp.where(eidx == e, wts_scr[...], 0.0), axis=1,
                   keepdims=True)
    acc_scr[...] += wcol * h2

    @pl.when(e == E - 1)
    def _finish_block():
        hnew = h_scr[...] + acc_scr[...]
        h_scr[...] = hnew

        @pl.when(i == NUM_BLOCKS - 1)
        def _head():
            hn_f = _ln(hnew, hlng[...], hlnb[...])
            lo = jax.lax.dot_general(
                hn_f, hw[...], (((1,), (1,)), ((), ())),
                preferred_element_type=jnp.float32) + hb[...]
            out[...] = lo
            aux[...] = aux_scr[...]


def _moe_stack(h0, p):
    full = lambda *shape: pl.BlockSpec(shape, lambda i, e: (0,) * len(shape))
    grid = (NUM_BLOCKS, E)
    out, aux = pl.pallas_call(
        _moe_kernel,
        grid=grid,
        in_specs=[
            full(BATCH, DIM),
            full(NUM_BLOCKS, 1, DIM),
            full(NUM_BLOCKS, 1, DIM),
            full(NUM_BLOCKS, E, DIM),
            full(NUM_BLOCKS, 1, E),
            full(NUM_BLOCKS * E, 1, HID),
            full(NUM_BLOCKS * E, 1, DIM),
            full(1, DIM),
            full(1, DIM),
            full(NUM_CLASSES, DIM),
            full(1, NUM_CLASSES),
            pl.BlockSpec((1, 1, HID, DIM), lambda i, e: (i, e, 0, 0)),
            pl.BlockSpec((1, 1, DIM, HID), lambda i, e: (i, e, 0, 0)),
        ],
        out_specs=[
            pl.BlockSpec((BATCH, NUM_CLASSES), lambda i, e: (0, 0)),
            pl.BlockSpec((1, 1), lambda i, e: (0, 0)),
        ],
        out_shape=[
            jax.ShapeDtypeStruct((BATCH, NUM_CLASSES), jnp.float32),
            jax.ShapeDtypeStruct((1, 1), jnp.float32),
        ],
        scratch_shapes=[
            pltpu.VMEM((BATCH, DIM), jnp.float32),
            pltpu.VMEM((BATCH, DIM), jnp.float32),
            pltpu.VMEM((BATCH, E), jnp.float32),
            pltpu.VMEM((BATCH, DIM), jnp.float32),
            pltpu.VMEM((1, 1), jnp.float32),
        ],
        interpret=INTERPRET,
    )(
        h0,
        p['ln_g'].reshape(NUM_BLOCKS, 1, DIM),
        p['ln_b'].reshape(NUM_BLOCKS, 1, DIM),
        p['router_w'],
        p['router_b'].reshape(NUM_BLOCKS, 1, E),
        p['b1'].reshape(NUM_BLOCKS * E, 1, HID),
        p['b2'].reshape(NUM_BLOCKS * E, 1, DIM),
        p['head_ln_g'].reshape(1, DIM),
        p['head_ln_b'].reshape(1, DIM),
        p['head_w'],
        p['head_b'].reshape(1, NUM_CLASSES),
        p['w1'],
        p['w2'],
    )
    return out, aux[0, 0]


def _stem(x, p):
    def conv(h, w, b):
        y = jax.lax.conv_general_dilated(
            h, w, (1, 1), 'SAME', dimension_numbers=('NCHW', 'OIHW', 'NCHW'))
        return y + b.reshape(1, -1, 1, 1)

    def bn(h, g, b, eps=1e-5):
        m = h.mean((0, 2, 3), keepdims=True)
        v = ((h - m) ** 2).mean((0, 2, 3), keepdims=True)
        return (h - m) / jnp.sqrt(v + eps) * g.reshape(1, -1, 1, 1) + \
            b.reshape(1, -1, 1, 1)

    g = lambda t: jax.nn.gelu(t, approximate=False)
    h = g(bn(conv(x, p['conv1_w'], p['conv1_b']), p['bn1_g'], p['bn1_b']))
    h = g(bn(conv(h, p['conv2_w'], p['conv2_b']), p['bn2_g'], p['bn2_b']))
    B, C, H, W = h.shape
    h = h.reshape(B, C, 4, H // 4, 4, W // 4).mean(axis=(3, 5))
    h = h.reshape(B, C * 16)
    h = g(h @ p['fc_w'].T + p['fc_b'])
    return h


def kernel(x, params):
    h0 = _stem(x, params)
    return _moe_stack(h0, params)
